# baseline (device time: 154509 ns/iter reference)
import jax
import jax.numpy as jnp
from jax import lax
from jax.experimental import pallas as pl
from jax.experimental.pallas import tpu as pltpu

N_DEV = 4
HQ = 8
DH = 128
DA = DH + 8
SCALE = 0.08838834764831843
EXP_OFF = 8.0


def kernel(x, Wq, Wo, K_ext, V_ext):
    Sq = x.shape[1]
    D = x.shape[2]
    Skv = K_ext.shape[1]

    kb = K_ext[0].reshape(Skv, D).astype(jnp.bfloat16)
    v0 = V_ext[0].astype(jnp.bfloat16).transpose(1, 0, 2)
    v_aug = jnp.concatenate(
        [v0,
         jnp.ones((HQ, Skv, 1), jnp.bfloat16),
         jnp.zeros((HQ, Skv, DA - DH - 1), jnp.bfloat16)],
        axis=2)

    def body(x_ref, wq_ref, wo_ref, k_ref, v_ref, out_ref,
             q_buf, acc_buf, attn_scr,
             q_send, q_recv, a_send, a_recv):
        my = lax.axis_index("i")
        right = lax.rem(my + 1, N_DEV)
        left = lax.rem(my + N_DEV - 1, N_DEV)

        barrier = pltpu.get_barrier_semaphore()
        for nbr in (left, right):
            pl.semaphore_signal(barrier, inc=1, device_id=(nbr,),
                                device_id_type=pl.DeviceIdType.MESH)
        pl.semaphore_wait(barrier, 2)

        def q_rdma(src_slot, dst_slot, dev):
            return pltpu.make_async_remote_copy(
                src_ref=q_buf.at[src_slot],
                dst_ref=q_buf.at[dst_slot],
                send_sem=q_send.at[src_slot],
                recv_sem=q_recv.at[dst_slot],
                device_id=(dev,),
                device_id_type=pl.DeviceIdType.MESH,
            )

        def a_rdma(src_slot, dst_slot, h, dev):
            return pltpu.make_async_remote_copy(
                src_ref=acc_buf.at[src_slot, h],
                dst_ref=acc_buf.at[dst_slot, h],
                send_sem=a_send.at[src_slot, h],
                recv_sem=a_recv.at[dst_slot, h],
                device_id=(dev,),
                device_id_type=pl.DeviceIdType.MESH,
            )

        def flash_head(slot, h, first):
            qh = q_buf[slot, :, h * DH:(h + 1) * DH]
            s = lax.dot_general(
                qh, k_ref[:, h * DH:(h + 1) * DH],
                (((1,), (1,)), ((), ())),
                preferred_element_type=jnp.float32)
            p = jnp.exp(s - EXP_OFF).astype(jnp.bfloat16)
            pv = lax.dot_general(
                p, v_ref[h], (((1,), (0,)), ((), ())),
                preferred_element_type=jnp.float32)
            if first:
                acc_buf[slot, h] = pv
            else:
                acc_buf[slot, h] = acc_buf[slot, h] + pv

        def send_head(step, h):
            a_rdma(step, (step + 1) % N_DEV, h, right).start()

        def wait_recv_head(slot, h):
            a_rdma(slot, slot, h, left).wait_recv()

        q = lax.dot_general(
            x_ref[:, :].astype(jnp.bfloat16),
            wq_ref[:, :].astype(jnp.bfloat16),
            (((1,), (0,)), ((), ())),
            preferred_element_type=jnp.float32)
        q_buf[0, :, :] = (q * SCALE).astype(jnp.bfloat16)
        q_rdma(0, 1, right).start()
        for hp in range(0, HQ, 2):
            flash_head(0, hp, first=True)
            flash_head(0, hp + 1, first=True)
            send_head(0, hp)
            send_head(0, hp + 1)

        for step in (1, 2, 3):
            q_rdma(step, step, left).wait_recv()
            if step < 3:
                q_rdma(step, step + 1, right).start()
            for hp in range(0, HQ, 2):
                wait_recv_head(step, hp)
                wait_recv_head(step, hp + 1)
                flash_head(step, hp, first=False)
                flash_head(step, hp + 1, first=False)
                send_head(step, hp)
                send_head(step, hp + 1)

        for h in range(HQ):
            wait_recv_head(0, h)
            l = acc_buf[0, h, :, DH:DH + 1]
            attn_scr[:, h * DH:(h + 1) * DH] = (
                acc_buf[0, h, :, 0:DH] / l).astype(jnp.bfloat16)
        out_ref[:, :] = lax.dot_general(
            attn_scr[:, :], wo_ref[:, :].astype(jnp.bfloat16),
            (((1,), (0,)), ((), ())),
            preferred_element_type=jnp.float32)

        for step in range(N_DEV):
            dst = (step + 1) % N_DEV
            if step < 3:
                q_rdma(step, dst, right).wait_send()
            for h in range(HQ):
                a_rdma(step, dst, h, right).wait_send()

    out = pl.pallas_call(
        body,
        out_shape=jax.ShapeDtypeStruct((Sq, D), jnp.float32),
        in_specs=[pl.BlockSpec(memory_space=pltpu.VMEM)] * 5,
        out_specs=pl.BlockSpec(memory_space=pltpu.VMEM),
        scratch_shapes=[
            pltpu.VMEM((N_DEV, Sq, D), jnp.bfloat16),
            pltpu.VMEM((N_DEV, HQ, Sq, DA), jnp.float32),
            pltpu.VMEM((Sq, D), jnp.bfloat16),
            pltpu.SemaphoreType.DMA((N_DEV,)),
            pltpu.SemaphoreType.DMA((N_DEV,)),
            pltpu.SemaphoreType.DMA((N_DEV, HQ)),
            pltpu.SemaphoreType.DMA((N_DEV, HQ)),
        ],
        compiler_params=pltpu.CompilerParams(
            collective_id=0, vmem_limit_bytes=100 * 1024 * 1024),
    )(x[0], Wq, Wo, kb, v_aug)

    return out.reshape(1, Sq, D)


# device time: 145682 ns/iter; 1.0606x vs baseline; 1.0606x over previous
import jax
import jax.numpy as jnp
from jax import lax
from jax.experimental import pallas as pl
from jax.experimental.pallas import tpu as pltpu

N_DEV = 4
HQ = 8
DH = 128
SCALE = 0.08838834764831843
EXP_OFF = 8.0


def kernel(x, Wq, Wo, K_ext, V_ext):
    Sq = x.shape[1]
    D = x.shape[2]
    Skv = K_ext.shape[1]

    kb = K_ext[0].reshape(Skv, D).astype(jnp.bfloat16)
    vb = V_ext[0].reshape(Skv, D).astype(jnp.bfloat16)

    def body(x_ref, wq_ref, wo_ref, k_ref, v_ref, out_ref,
             q_buf, acc_buf, st_buf, attn_scr,
             q_send, q_recv, a_send, a_recv, s_send, s_recv):
        my = lax.axis_index("i")
        right = lax.rem(my + 1, N_DEV)
        left = lax.rem(my + N_DEV - 1, N_DEV)

        barrier = pltpu.get_barrier_semaphore()
        for nbr in (left, right):
            pl.semaphore_signal(barrier, inc=1, device_id=(nbr,),
                                device_id_type=pl.DeviceIdType.MESH)
        pl.semaphore_wait(barrier, 2)

        def q_rdma(src_slot, dst_slot, dev):
            return pltpu.make_async_remote_copy(
                src_ref=q_buf.at[src_slot],
                dst_ref=q_buf.at[dst_slot],
                send_sem=q_send.at[src_slot],
                recv_sem=q_recv.at[dst_slot],
                device_id=(dev,),
                device_id_type=pl.DeviceIdType.MESH,
            )

        def head_rdma(buf, ss, rs, src_slot, dst_slot, h, dev):
            return pltpu.make_async_remote_copy(
                src_ref=buf.at[src_slot, h],
                dst_ref=buf.at[dst_slot, h],
                send_sem=ss.at[src_slot, h],
                recv_sem=rs.at[dst_slot, h],
                device_id=(dev,),
                device_id_type=pl.DeviceIdType.MESH,
            )

        def flash_head(slot, h, first):
            qh = q_buf[slot, :, h * DH:(h + 1) * DH]
            s = lax.dot_general(
                qh, k_ref[:, h * DH:(h + 1) * DH],
                (((1,), (1,)), ((), ())),
                preferred_element_type=jnp.float32)
            p32 = jnp.exp(s - EXP_OFF)
            ps = jnp.sum(p32, axis=1, keepdims=True)
            pv = lax.dot_general(
                p32.astype(jnp.bfloat16), v_ref[:, h * DH:(h + 1) * DH],
                (((1,), (0,)), ((), ())),
                preferred_element_type=jnp.float32)
            if first:
                acc_buf[slot, h] = pv
                st_buf[slot, h, :, 0:1] = ps
            else:
                acc_buf[slot, h] = acc_buf[slot, h] + pv
                st_buf[slot, h, :, 0:1] = st_buf[slot, h, :, 0:1] + ps

        def send_head(step, h):
            dst = (step + 1) % N_DEV
            head_rdma(acc_buf, a_send, a_recv, step, dst, h, right).start()
            head_rdma(st_buf, s_send, s_recv, step, dst, h, right).start()

        def wait_recv_head(slot, h):
            head_rdma(acc_buf, a_send, a_recv, slot, slot, h, left).wait_recv()
            head_rdma(st_buf, s_send, s_recv, slot, slot, h, left).wait_recv()

        q = lax.dot_general(
            x_ref[:, :].astype(jnp.bfloat16),
            wq_ref[:, :].astype(jnp.bfloat16),
            (((1,), (0,)), ((), ())),
            preferred_element_type=jnp.float32)
        q_buf[0, :, :] = (q * SCALE).astype(jnp.bfloat16)
        q_rdma(0, 1, right).start()
        for hp in range(0, HQ, 2):
            flash_head(0, hp, first=True)
            flash_head(0, hp + 1, first=True)
            send_head(0, hp)
            send_head(0, hp + 1)

        for step in (1, 2, 3):
            q_rdma(step, step, left).wait_recv()
            if step < 3:
                q_rdma(step, step + 1, right).start()
            for hp in range(0, HQ, 2):
                wait_recv_head(step, hp)
                wait_recv_head(step, hp + 1)
                flash_head(step, hp, first=False)
                flash_head(step, hp + 1, first=False)
                send_head(step, hp)
                send_head(step, hp + 1)

        for h in range(HQ):
            wait_recv_head(0, h)
            l = st_buf[0, h, :, 0:1]
            attn_scr[:, h * DH:(h + 1) * DH] = (
                acc_buf[0, h] / l).astype(jnp.bfloat16)
        out_ref[:, :] = lax.dot_general(
            attn_scr[:, :], wo_ref[:, :].astype(jnp.bfloat16),
            (((1,), (0,)), ((), ())),
            preferred_element_type=jnp.float32)

        for step in range(N_DEV):
            dst = (step + 1) % N_DEV
            if step < 3:
                q_rdma(step, dst, right).wait_send()
            for h in range(HQ):
                head_rdma(acc_buf, a_send, a_recv, step, dst, h,
                          right).wait_send()
                head_rdma(st_buf, s_send, s_recv, step, dst, h,
                          right).wait_send()

    out = pl.pallas_call(
        body,
        out_shape=jax.ShapeDtypeStruct((Sq, D), jnp.float32),
        in_specs=[pl.BlockSpec(memory_space=pltpu.VMEM)] * 5,
        out_specs=pl.BlockSpec(memory_space=pltpu.VMEM),
        scratch_shapes=[
            pltpu.VMEM((N_DEV, Sq, D), jnp.bfloat16),
            pltpu.VMEM((N_DEV, HQ, Sq, DH), jnp.float32),
            pltpu.VMEM((N_DEV, HQ, Sq, 1), jnp.float32),
            pltpu.VMEM((Sq, D), jnp.bfloat16),
            pltpu.SemaphoreType.DMA((N_DEV,)),
            pltpu.SemaphoreType.DMA((N_DEV,)),
            pltpu.SemaphoreType.DMA((N_DEV, HQ)),
            pltpu.SemaphoreType.DMA((N_DEV, HQ)),
            pltpu.SemaphoreType.DMA((N_DEV, HQ)),
            pltpu.SemaphoreType.DMA((N_DEV, HQ)),
        ],
        compiler_params=pltpu.CompilerParams(
            collective_id=0, vmem_limit_bytes=100 * 1024 * 1024),
    )(x[0], Wq, Wo, kb, vb)

    return out.reshape(1, Sq, D)
